# Initial kernel scaffold; baseline (speedup 1.0000x reference)
import jax
import jax.numpy as jnp
from jax.experimental import pallas as pl


def kernel(proposal_feature, proposal_deltas, proposal_scale, roi_feature, roi_deltas, roi_scale, roi_class, W_rpn_f, b_rpn_f, W_rpn_d, b_rpn_d, W_rpn_s, b_rpn_s, W_roi_f, b_roi_f, W_roi_d, b_roi_d, W_roi_s, b_roi_s):
    raise NotImplementedError("write your pallas kernel here")


# speedup vs baseline: 2.1978x; 2.1978x over previous
import jax
import jax.numpy as jnp
from jax import lax
from jax.experimental import pallas as pl
from jax.experimental.pallas import tpu as pltpu

_N_CLS = 80
_BR = 2000


def _dot0(a, b):
    return lax.dot_general(a, b, (((0,), (0,)), ((), ())),
                           preferred_element_type=jnp.float32)


def _body(pf_ref, pd_ref, ps_ref, rf_ref, rd_ref, rs_ref, rc_ref,
          wrf_ref, brf_ref, wrd_ref, brd_ref, wrs_ref, brs_ref,
          qrf_ref, crf_ref, qrd_ref, crd_ref, qrs_ref, crs_ref,
          o_rpna_ref, o_rpnb_ref, o_cls_ref, o_bbox_ref, o_sup_ref,
          seg_f, seg_d, seg_s, cnt, psf, psd, pss):
    i = pl.program_id(0)
    nb = pl.num_programs(0)
    n_total = nb * _BR

    @pl.when(i == 0)
    def _init():
        seg_f[...] = jnp.zeros_like(seg_f)
        seg_d[...] = jnp.zeros_like(seg_d)
        seg_s[...] = jnp.zeros_like(seg_s)
        cnt[...] = jnp.zeros_like(cnt)
        psf[...] = jnp.zeros_like(psf)
        psd[...] = jnp.zeros_like(psd)
        pss[...] = jnp.zeros_like(pss)

    psf[...] += jnp.sum(pf_ref[...], axis=0, keepdims=True)
    psd[...] += jnp.sum(pd_ref[...], axis=0, keepdims=True)
    pss[...] += jnp.sum(ps_ref[...], axis=0, keepdims=True)

    cls = rc_ref[...]
    onehot = (cls == lax.broadcasted_iota(jnp.int32, (_BR, _N_CLS), 1)
              ).astype(jnp.float32)
    seg_f[...] += _dot0(onehot, rf_ref[...])
    seg_d[...] += _dot0(onehot, rd_ref[...])
    seg_s[...] += _dot0(onehot, rs_ref[...])
    cnt[...] += jnp.sum(onehot, axis=0, keepdims=True).reshape(_N_CLS, 1)

    @pl.when(i == nb - 1)
    def _final():
        inv_n = 1.0 / n_total
        b_rpn = (brf_ref[...] + brd_ref[...] + brs_ref[...]).reshape(1, -1)
        rpn_w = (jnp.dot(psf[...], wrf_ref[...],
                         preferred_element_type=jnp.float32)
                 + jnp.dot(psd[...], wrd_ref[...],
                           preferred_element_type=jnp.float32)
                 + pss[0, 0] * wrs_ref[...]) * inv_n + b_rpn
        o_rpna_ref[...] = rpn_w[0, :128]
        o_rpnb_ref[...] = rpn_w[0, 128:]

        b_roi = (crf_ref[...] + crd_ref[...] + crs_ref[...]).reshape(1, -1)
        sums = (jnp.dot(seg_f[...], qrf_ref[...],
                        preferred_element_type=jnp.float32)
                + jnp.dot(seg_d[...], qrd_ref[...],
                          preferred_element_type=jnp.float32)
                + jnp.dot(seg_s[...], qrs_ref[...],
                          preferred_element_type=jnp.float32)
                + cnt[...] * b_roi)

        tot_f = jnp.sum(seg_f[...], axis=0, keepdims=True)
        tot_d = jnp.sum(seg_d[...], axis=0, keepdims=True)
        tot_s = jnp.sum(seg_s[...], axis=0, keepdims=True)
        bbox = (jnp.dot(tot_f, qrf_ref[...],
                        preferred_element_type=jnp.float32)
                + jnp.dot(tot_d, qrd_ref[...],
                          preferred_element_type=jnp.float32)
                + tot_s[0, 0] * qrs_ref[...]) * inv_n + b_roi
        o_bbox_ref[...] = bbox[:, 128:]

        c_int = lax.broadcasted_iota(jnp.int32, (_N_CLS, _N_CLS), 0)
        j_int = lax.broadcasted_iota(jnp.int32, (_N_CLS, _N_CLS), 1)
        c_idx = c_int.astype(jnp.float32)
        j_idx = j_int.astype(jnp.float32)
        m = (cnt[...] > 0).astype(jnp.float32)
        tri = (j_idx <= c_idx).astype(jnp.float32)
        rank = jnp.dot(tri, m, preferred_element_type=jnp.float32) - 1.0
        npres = jnp.sum(m)
        present = jnp.logical_and(m > 0, rank == j_idx)
        fill = jnp.logical_and(j_idx >= npres, c_idx == 0)
        h = jnp.logical_or(present, fill).astype(jnp.float32)

        o_sup_ref[...] = jnp.sum(h * c_idx, axis=0).astype(jnp.int32)
        g_sums = _dot0(h, sums)
        g_cnt = _dot0(h, cnt[...])
        means = g_sums / g_cnt
        o_cls_ref[...] = means[:, :128]


def kernel(proposal_feature, proposal_deltas, proposal_scale, roi_feature,
           roi_deltas, roi_scale, roi_class, W_rpn_f, b_rpn_f, W_rpn_d,
           b_rpn_d, W_rpn_s, b_rpn_s, W_roi_f, b_roi_f, W_roi_d, b_roi_d,
           W_roi_s, b_roi_s):
    n = proposal_feature.shape[0]
    nb = n // _BR
    d_rpn = proposal_feature.shape[1]
    d_roi = roi_feature.shape[1]

    ps2 = proposal_scale.reshape(n, 1)
    rs2 = roi_scale.reshape(n, 1)
    rc2 = roi_class.reshape(n, 1)

    row = lambda shape: pl.BlockSpec(shape, lambda i: (i, 0))
    whole = lambda a: pl.BlockSpec(a.shape, lambda i: (0,) * a.ndim)

    out_shapes = (
        jax.ShapeDtypeStruct((d_rpn,), jnp.float32),
        jax.ShapeDtypeStruct((d_rpn,), jnp.float32),
        jax.ShapeDtypeStruct((_N_CLS, d_roi), jnp.float32),
        jax.ShapeDtypeStruct((1, d_roi), jnp.float32),
        jax.ShapeDtypeStruct((_N_CLS,), jnp.int32),
    )
    out_specs = (
        pl.BlockSpec((d_rpn,), lambda i: (0,)),
        pl.BlockSpec((d_rpn,), lambda i: (0,)),
        pl.BlockSpec((_N_CLS, d_roi), lambda i: (0, 0)),
        pl.BlockSpec((1, d_roi), lambda i: (0, 0)),
        pl.BlockSpec((_N_CLS,), lambda i: (0,)),
    )
    in_specs = [
        row((_BR, d_rpn)), row((_BR, 4)), row((_BR, 1)),
        row((_BR, d_roi)), row((_BR, 4)), row((_BR, 1)), row((_BR, 1)),
        whole(W_rpn_f), whole(b_rpn_f), whole(W_rpn_d), whole(b_rpn_d),
        whole(W_rpn_s), whole(b_rpn_s), whole(W_roi_f), whole(b_roi_f),
        whole(W_roi_d), whole(b_roi_d), whole(W_roi_s), whole(b_roi_s),
    ]
    scratch_shapes = [
        pltpu.VMEM((_N_CLS, d_roi), jnp.float32),
        pltpu.VMEM((_N_CLS, 4), jnp.float32),
        pltpu.VMEM((_N_CLS, 1), jnp.float32),
        pltpu.VMEM((_N_CLS, 1), jnp.float32),
        pltpu.VMEM((1, d_rpn), jnp.float32),
        pltpu.VMEM((1, 4), jnp.float32),
        pltpu.VMEM((1, 1), jnp.float32),
    ]

    return pl.pallas_call(
        _body,
        grid=(nb,),
        in_specs=in_specs,
        out_specs=out_specs,
        out_shape=out_shapes,
        scratch_shapes=scratch_shapes,
    )(proposal_feature, proposal_deltas, ps2, roi_feature, roi_deltas,
      rs2, rc2, W_rpn_f, b_rpn_f, W_rpn_d, b_rpn_d, W_rpn_s, b_rpn_s,
      W_roi_f, b_roi_f, W_roi_d, b_roi_d, W_roi_s, b_roi_s)
